# Initial kernel scaffold; baseline (speedup 1.0000x reference)
import jax
import jax.numpy as jnp
from jax.experimental import pallas as pl


def kernel(input_features, edge_i2t, edge_t2i, embed_item, pre_Wi, pre_bi, pre_Wh, pre_bh, pre_Wo, pre_bo, c1_W_i2t, c1_b_i2t, c1_W_t2i, c1_b_t2i, c2_W_i2t, c2_b_i2t, c2_W_t2i, c2_b_t2i, c3_W_i2t, c3_b_i2t, c3_W_t2i, c3_b_t2i, post_Wi, post_bi, post_Wh, post_bh, post_Wo, post_bo):
    raise NotImplementedError("write your pallas kernel here")


# speedup vs baseline: 5.5234x; 5.5234x over previous
import functools

import jax
import jax.numpy as jnp
from jax import lax
from jax.experimental import pallas as pl
from jax.experimental.pallas import tpu as pltpu
from jax.experimental.pallas import tpu_sc as plsc

N_NODE = 10000
E = 320000
NC = 2
NS = 16
NW = NC * NS
EPW = E // NW
CHUNK = 80
RPT = EPW // CHUNK
ROWS = E // CHUNK
NPT = N_NODE // NS
NZB = NPT // 5

_mesh = plsc.VectorSubcoreMesh(core_axis_name="c", subcore_axis_name="s")
_sc_params = pltpu.CompilerParams(needs_layout_passes=False,
                                  use_tc_tiling_on_sc=False)


@functools.partial(
    pl.kernel, mesh=_mesh, compiler_params=_sc_params,
    out_type=jax.ShapeDtypeStruct((4, NW, N_NODE), jnp.float32),
    scratch_types=[
        pltpu.VMEM((N_NODE,), jnp.float32),
        pltpu.VMEM((N_NODE,), jnp.float32),
        pltpu.VMEM((N_NODE,), jnp.float32),
        pltpu.VMEM((N_NODE,), jnp.float32),
        pltpu.VMEM((EPW,), jnp.int32),
    ],
)
def _degree_hist(src_it, dst_it, src_ti, dst_ti, out, h0, h1, h2, h3, idxbuf):
    c = lax.axis_index("c")
    s = lax.axis_index("s")
    wid = s * NC + c
    z16 = jnp.zeros((16,), jnp.float32)
    hists = (h0, h1, h2, h3)

    def zbody(i, _):
        for h in hists:
            h[pl.ds(i * 16, 16)] = z16
        return 0
    lax.fori_loop(0, N_NODE // 16, zbody, 0)

    ones16 = jnp.ones((16,), jnp.float32)
    for k, edges in enumerate((src_it, dst_it, src_ti, dst_ti)):
        pltpu.sync_copy(edges.at[pl.ds(wid * EPW, EPW)], idxbuf)
        h = hists[k]

        def abody(i, _):
            idx16 = idxbuf[pl.ds(i * 16, 16)]
            plsc.addupdate_scatter(h, [idx16], ones16)
            return 0
        lax.fori_loop(0, EPW // 16, abody, 0)
        pltpu.sync_copy(h, out.at[k].at[wid])


def _make_agg(d):
    @functools.partial(
        pl.kernel, mesh=_mesh, compiler_params=_sc_params,
        out_type=jax.ShapeDtypeStruct((NC * N_NODE, d), jnp.float32),
        scratch_types=[
            pltpu.VMEM((RPT, CHUNK), jnp.int32),
            pltpu.VMEM((RPT, CHUNK), jnp.int32),
            pltpu.VMEM((CHUNK, d), jnp.float32),
            pltpu.VMEM((NZB, d), jnp.float32),
            pltpu.VMEM_SHARED((N_NODE, d), jnp.float32),
            pltpu.SemaphoreType.DMA,
        ],
    )
    def _agg(x, src2, dst2, out, idx_s, idx_d, rows, zbuf, acc, sem):
        c = lax.axis_index("c")
        s = lax.axis_index("s")
        wid = s * NC + c
        z16 = jnp.zeros((16,), jnp.float32)

        def zbody(r, _):
            for kcol in range(d // 16):
                zbuf[r, pl.ds(kcol * 16, 16)] = z16
            return 0
        lax.fori_loop(0, NZB, zbody, 0)
        for t in range(5):
            pltpu.sync_copy(zbuf, acc.at[pl.ds(s * NPT + t * NZB, NZB)])
        plsc.subcore_barrier()

        pltpu.sync_copy(src2.at[pl.ds(wid * RPT, RPT)], idx_s)
        pltpu.sync_copy(dst2.at[pl.ds(wid * RPT, RPT)], idx_d)

        def ebody(j, _):
            pltpu.async_copy(x.at[idx_s.at[j]], rows, sem).wait()
            pltpu.sync_copy(rows, acc.at[idx_d.at[j]], add=True)
            return 0
        lax.fori_loop(0, RPT, ebody, 0)
        plsc.subcore_barrier()

        for t in range(5):
            off = s * NPT + t * NZB
            pltpu.sync_copy(acc.at[pl.ds(off, NZB)], zbuf)
            pltpu.sync_copy(zbuf, out.at[pl.ds(c * N_NODE + off, NZB)])

    return _agg


_agg128 = _make_agg(128)
_agg64 = _make_agg(64)
_agg16 = _make_agg(16)


BM = 2000


def _scales_body(hp_ref, o_ref):
    deg = jnp.sum(hp_ref[...], axis=1)
    o_ref[...] = lax.rsqrt(jnp.maximum(deg, 1.0))


def _scales(hp):
    return pl.pallas_call(
        _scales_body,
        out_shape=jax.ShapeDtypeStruct((4, N_NODE), jnp.float32),
    )(hp)


def _mm_body(x_ref, s_ref, w_ref, o_ref):
    o_ref[...] = jnp.dot(x_ref[...] * s_ref[...], w_ref[...],
                         preferred_element_type=jnp.float32)


def _scaled_mm(x, sc, w):
    m, din = x.shape
    dout = w.shape[1]
    return pl.pallas_call(
        _mm_body,
        grid=(m // BM,),
        in_specs=[pl.BlockSpec((BM, din), lambda i: (i, 0)),
                  pl.BlockSpec((BM, 1), lambda i: (i, 0)),
                  pl.BlockSpec((din, dout), lambda i: (0, 0))],
        out_specs=pl.BlockSpec((BM, dout), lambda i: (i, 0)),
        out_shape=jax.ShapeDtypeStruct((m, dout), jnp.float32),
    )(x, sc, w)


def _post_body(relu, p_ref, s_ref, b_ref, o_ref):
    v = (p_ref[0] + p_ref[1]) * s_ref[...] + b_ref[...]
    if relu:
        v = jnp.maximum(v, 0.0)
    o_ref[...] = v


def _post(p, sc, b, relu):
    d = p.shape[-1]
    return pl.pallas_call(
        functools.partial(_post_body, relu),
        grid=(N_NODE // BM,),
        in_specs=[pl.BlockSpec((2, BM, d), lambda i: (0, i, 0)),
                  pl.BlockSpec((BM, 1), lambda i: (i, 0)),
                  pl.BlockSpec((1, d), lambda i: (0, 0))],
        out_specs=pl.BlockSpec((BM, d), lambda i: (i, 0)),
        out_shape=jax.ShapeDtypeStruct((N_NODE, d), jnp.float32),
    )(p, sc, b)


def _ff_body(x_ref, wi, bi, wh, bh, wo, bo, o_ref):
    h = jnp.maximum(jnp.dot(x_ref[...], wi[...],
                            preferred_element_type=jnp.float32) + bi[...], 0.0)
    h = jnp.maximum(jnp.dot(h, wh[...],
                            preferred_element_type=jnp.float32) + bh[...], 0.0)
    o_ref[...] = jnp.dot(h, wo[...],
                         preferred_element_type=jnp.float32) + bo[...]


def _ff(x, wi, bi, wh, bh, wo, bo):
    m, din = x.shape
    dh = wi.shape[1]
    dout = wo.shape[1]
    return pl.pallas_call(
        _ff_body,
        grid=(m // BM,),
        in_specs=[pl.BlockSpec((BM, din), lambda i: (i, 0)),
                  pl.BlockSpec((din, dh), lambda i: (0, 0)),
                  pl.BlockSpec((1, dh), lambda i: (0, 0)),
                  pl.BlockSpec((dh, dh), lambda i: (0, 0)),
                  pl.BlockSpec((1, dh), lambda i: (0, 0)),
                  pl.BlockSpec((dh, dout), lambda i: (0, 0)),
                  pl.BlockSpec((1, dout), lambda i: (0, 0))],
        out_specs=pl.BlockSpec((BM, dout), lambda i: (i, 0)),
        out_shape=jax.ShapeDtypeStruct((m, dout), jnp.float32),
    )(x, wi, bi.reshape(1, -1), wh, bh.reshape(1, -1), wo, bo.reshape(1, -1))


def kernel(input_features, edge_i2t, edge_t2i, embed_item,
           pre_Wi, pre_bi, pre_Wh, pre_bh, pre_Wo, pre_bo,
           c1_W_i2t, c1_b_i2t, c1_W_t2i, c1_b_t2i,
           c2_W_i2t, c2_b_i2t, c2_W_t2i, c2_b_t2i,
           c3_W_i2t, c3_b_i2t, c3_W_t2i, c3_b_t2i,
           post_Wi, post_bi, post_Wh, post_bh, post_Wo, post_bo):
    src_it = edge_i2t[0].astype(jnp.int32)
    dst_it = edge_i2t[1].astype(jnp.int32)
    src_ti = edge_t2i[0].astype(jnp.int32)
    dst_ti = edge_t2i[1].astype(jnp.int32)

    hp = _degree_hist(src_it, dst_it, src_ti, dst_ti)
    s4 = _scales(hp)
    so_it = s4[0].reshape(-1, 1)
    si_it = s4[1].reshape(-1, 1)
    so_ti = s4[2].reshape(-1, 1)
    si_ti = s4[3].reshape(-1, 1)

    it_s2 = src_it.reshape(ROWS, CHUNK)
    it_d2 = dst_it.reshape(ROWS, CHUNK)
    ti_s2 = src_ti.reshape(ROWS, CHUNK)
    ti_d2 = dst_ti.reshape(ROWS, CHUNK)

    def agg(fn, y, s2, d2):
        p = fn(y, s2, d2)
        return p.reshape(2, N_NODE, -1)

    tgt = _ff(input_features.astype(jnp.float32),
              pre_Wi, pre_bi, pre_Wh, pre_bh, pre_Wo, pre_bo)

    y = _scaled_mm(embed_item, so_it, c1_W_i2t)
    h1_t = _post(agg(_agg128, y, it_s2, it_d2), si_it,
                 c1_b_i2t.reshape(1, -1), relu=True)
    y = _scaled_mm(tgt, so_ti, c1_W_t2i)
    h1_i = _post(agg(_agg128, y, ti_s2, ti_d2), si_ti,
                 c1_b_t2i.reshape(1, -1), relu=True)

    y = _scaled_mm(h1_i, so_it, c2_W_i2t)
    h2_t = _post(agg(_agg128, y, it_s2, it_d2), si_it,
                 c2_b_i2t.reshape(1, -1), relu=True)
    y = _scaled_mm(h1_t, so_ti, c2_W_t2i)
    h2_i = _post(agg(_agg128, y, ti_s2, ti_d2), si_ti,
                 c2_b_t2i.reshape(1, -1), relu=True)

    y = _scaled_mm(h2_i, so_it, c3_W_i2t)
    h3_t = _post(agg(_agg64, y, it_s2, it_d2), si_it,
                 c3_b_i2t.reshape(1, -1), relu=False)
    w3 = jnp.pad(c3_W_t2i, ((0, 0), (0, 15)))
    b3 = jnp.pad(c3_b_t2i, (0, 15)).reshape(1, -1)
    y = _scaled_mm(h2_t, so_ti, w3)
    h3_i = _post(agg(_agg16, y, ti_s2, ti_d2), si_ti, b3, relu=False)[:, :1]

    out_t = _ff(h3_t, post_Wi, post_bi, post_Wh, post_bh, post_Wo, post_bo)
    return (out_t, h3_i)
